# Initial kernel scaffold; baseline (speedup 1.0000x reference)
import jax
import jax.numpy as jnp
from jax.experimental import pallas as pl


def kernel(x, pos, batch, edge_index, params):
    raise NotImplementedError("write your pallas kernel here")


# speedup vs baseline: 1.0129x; 1.0129x over previous
import functools

import jax
import jax.numpy as jnp
import numpy as np
from jax import lax
from jax.experimental import pallas as pl
from jax.experimental.pallas import tpu as pltpu

N = 10000
E = 160000
DIM1, DIM2, DIM3 = 16, 32, 64
K3 = 8
NUM_CLASSES = 10
M = 1000
MP = 1024
POOL_RADIUS = 0.2
NODE_RADIUS = 1.0

_BITS = [((s >> 2) & 1, (s >> 1) & 1, s & 1) for s in range(8)]
_BITS_F = jnp.array(_BITS, dtype=jnp.float32)


def _elu(v):
    return jnp.where(v > 0, v, jnp.exp(v) - 1.0)


def _pooled_kernel(spos_ref, sposT_ref, sx_ref,
                   w6_ref, root6_ref, b6_ref, g6_ref, be6_ref,
                   w7_ref, root7_ref, b7_ref, g7_ref, be7_ref,
                   fcw_ref, fcb_ref, out_ref):
    CH = 256
    NCH = MP // CH
    inv2r = 1.0 / (2.0 * POOL_RADIUS)
    r2 = POOL_RADIUS * POOL_RADIUS

    sx = sx_ref[...]
    vmask_col = (lax.broadcasted_iota(jnp.int32, (MP, 1), 0) < M)

    def dense_msg(xfeat, w_ref):
        xwcat = jnp.concatenate(
            [jnp.dot(xfeat, w_ref[s], preferred_element_type=jnp.float32)
             for s in range(K3)], axis=0)
        msg_chunks = []
        deg_chunks = []
        for ci in range(NCH):
            i0 = ci * CH
            pxi = spos_ref[i0:i0 + CH, 0:1]
            pyi = spos_ref[i0:i0 + CH, 1:2]
            pzi = spos_ref[i0:i0 + CH, 2:3]
            dx = sposT_ref[0:1, :] - pxi
            dy = sposT_ref[1:2, :] - pyi
            dz = sposT_ref[2:3, :] - pzi
            d2 = dx * dx + dy * dy + dz * dz
            rowid = lax.broadcasted_iota(jnp.int32, (CH, MP), 0) + i0
            colid = lax.broadcasted_iota(jnp.int32, (CH, MP), 1)
            mask = (d2 < r2) & (rowid != colid)
            maskf = mask.astype(jnp.float32)
            ux = jnp.clip(dx * inv2r + 0.5, 0.0, 1.0)
            uy = jnp.clip(dy * inv2r + 0.5, 0.0, 1.0)
            uz = jnp.clip(dz * inv2r + 0.5, 0.0, 1.0)
            bs = []
            for (b0, b1_, b2_) in _BITS:
                w = (ux if b0 else 1.0 - ux)
                w = w * (uy if b1_ else 1.0 - uy)
                w = w * (uz if b2_ else 1.0 - uz)
                bs.append(w * maskf)
            b2cat = jnp.concatenate(bs, axis=1)
            msg_chunks.append(jnp.dot(b2cat, xwcat,
                                      preferred_element_type=jnp.float32))
            deg_chunks.append(jnp.sum(maskf, axis=1, keepdims=True))
        msg = jnp.concatenate(msg_chunks, axis=0)
        deg = jnp.concatenate(deg_chunks, axis=0)
        return msg, deg

    def bn(h, g_ref, be_ref):
        hm = jnp.where(vmask_col, h, 0.0)
        mu = jnp.sum(hm, axis=0, keepdims=True) * (1.0 / M)
        d = h - mu
        dm = jnp.where(vmask_col, d * d, 0.0)
        var = jnp.sum(dm, axis=0, keepdims=True) * (1.0 / M)
        return d * jax.lax.rsqrt(var + 1e-5) * g_ref[...] + be_ref[...]

    msg6, deg = dense_msg(sx, w6_ref)
    degc = jnp.maximum(deg, 1.0)
    h6 = msg6 / degc + jnp.dot(sx, root6_ref[...],
                               preferred_element_type=jnp.float32) + b6_ref[...]
    h6 = bn(_elu(h6), g6_ref, be6_ref)

    msg7, _ = dense_msg(h6, w7_ref)
    h7 = msg7 / degc + jnp.dot(h6, root7_ref[...],
                               preferred_element_type=jnp.float32) + b7_ref[...]
    h7 = bn(_elu(h7), g7_ref, be7_ref)

    h2 = h7 + sx
    g = jnp.max(jnp.where(vmask_col, h2, -jnp.inf), axis=0, keepdims=True)
    logits = jnp.dot(g, fcw_ref[...],
                     preferred_element_type=jnp.float32) + fcb_ref[...]
    out_ref[...] = jnp.zeros((8, 128), jnp.float32)
    out_ref[0:1, 0:NUM_CLASSES] = logits


def _pooled_stage(spos, sx, p6, bn6, p7, bn7, fc_w, fc_b):
    sposp = jnp.concatenate(
        [spos, jnp.zeros((M, 1), jnp.float32)], axis=1)
    sposp = jnp.concatenate(
        [sposp, jnp.full((MP - M, 4), 1e6, jnp.float32)], axis=0)
    sxp = jnp.concatenate([sx, jnp.zeros((MP - M, DIM3), sx.dtype)], axis=0)
    out = pl.pallas_call(
        _pooled_kernel,
        out_shape=jax.ShapeDtypeStruct((8, 128), jnp.float32),
    )(sposp, sposp.T, sxp,
      p6['W'], p6['root'], p6['b'][None, :], bn6['gamma'][None, :],
      bn6['beta'][None, :],
      p7['W'], p7['root'], p7['b'][None, :], bn7['gamma'][None, :],
      bn7['beta'][None, :],
      fc_w, fc_b[None, :])
    return out[0:1, 0:NUM_CLASSES]


def _basis_edges(u):
    u = jnp.clip(u, 0.0, 1.0)
    ue = u[..., None, :]
    return jnp.prod(jnp.where(_BITS_F > 0, ue, 1.0 - ue), axis=-1)


def _conv_edges(x, src, dst, b, p, n_nodes):
    xs = x[src]
    msg = jnp.zeros((xs.shape[0], p['W'].shape[2]), dtype=x.dtype)
    for k in range(K3):
        msg = msg + b[:, k:k + 1] * (xs @ p['W'][k])
    agg = jax.ops.segment_sum(msg, dst, num_segments=n_nodes)
    cnt = jax.ops.segment_sum(jnp.ones((xs.shape[0],), dtype=x.dtype), dst,
                              num_segments=n_nodes)
    return agg / jnp.maximum(cnt, 1.0)[:, None] + x @ p['root'] + p['b']


def _bn_full(x, p, eps=1e-5):
    mu = x.mean(axis=0)
    var = x.var(axis=0)
    return (x - mu) / jnp.sqrt(var + eps) * p['gamma'] + p['beta']


def _fps_jax(pos, m):
    n = pos.shape[0]
    idxs = jnp.zeros((m,), dtype=jnp.int32)
    dist = jnp.full((n,), jnp.inf, dtype=pos.dtype)

    def body(i, carry):
        dist, idxs = carry
        last = pos[idxs[i]]
        d = jnp.sum((pos - last) ** 2, axis=1)
        dist = jnp.minimum(dist, d)
        idxs = idxs.at[i + 1].set(jnp.argmax(dist).astype(jnp.int32))
        return dist, idxs

    dist, idxs = jax.lax.fori_loop(0, m - 1, body, (dist, idxs))
    return idxs


def kernel(x, pos, batch, edge_index, params):
    src, dst = edge_index[0], edge_index[1]
    u1 = (pos[src] - pos[dst]) / (2.0 * NODE_RADIUS) + 0.5
    b1 = _basis_edges(u1)
    h = params['emb'][x]
    h = _bn_full(jax.nn.elu(_conv_edges(h, src, dst, b1, params['conv1'], N)),
                 params['bn1'])
    h = _bn_full(jax.nn.elu(_conv_edges(h, src, dst, b1, params['conv2'], N)),
                 params['bn2'])
    h_sc = h
    h = _bn_full(jax.nn.elu(_conv_edges(h, src, dst, b1, params['conv3'], N)),
                 params['bn3'])
    h = _bn_full(jax.nn.elu(_conv_edges(h, src, dst, b1, params['conv4'], N)),
                 params['bn4'])
    h = h + h_sc
    h = _bn_full(jax.nn.elu(_conv_edges(h, src, dst, b1, params['conv5'], N)),
                 params['bn5'])
    idx = _fps_jax(lax.stop_gradient(pos), M)
    sx = h[idx]
    spos = pos[idx]
    return _pooled_stage(spos, sx, params['conv6'], params['bn6'],
                         params['conv7'], params['bn7'],
                         params['fc_w'], params['fc_b'])


# speedup vs baseline: 2.1583x; 2.1308x over previous
import functools

import jax
import jax.numpy as jnp
import numpy as np
from jax import lax
from jax.experimental import pallas as pl
from jax.experimental.pallas import tpu as pltpu

N = 10000
E = 160000
DIM1, DIM2, DIM3 = 16, 32, 64
K3 = 8
NUM_CLASSES = 10
M = 1000
MP = 1024
POOL_RADIUS = 0.2
NODE_RADIUS = 1.0

_BITS = [((s >> 2) & 1, (s >> 1) & 1, s & 1) for s in range(8)]
_BITS_F = jnp.array(_BITS, dtype=jnp.float32)


def _elu(v):
    return jnp.where(v > 0, v, jnp.exp(v) - 1.0)


def _pooled_kernel(spos_ref, sposT_ref, sx_ref,
                   w6_ref, root6_ref, b6_ref, g6_ref, be6_ref,
                   w7_ref, root7_ref, b7_ref, g7_ref, be7_ref,
                   fcw_ref, fcb_ref, out_ref):
    CH = 256
    NCH = MP // CH
    inv2r = 1.0 / (2.0 * POOL_RADIUS)
    r2 = POOL_RADIUS * POOL_RADIUS

    sx = sx_ref[...]
    vmask_col = (lax.broadcasted_iota(jnp.int32, (MP, 1), 0) < M)

    def dense_msg(xfeat, w_ref):
        xwcat = jnp.concatenate(
            [jnp.dot(xfeat, w_ref[s], preferred_element_type=jnp.float32)
             for s in range(K3)], axis=0)
        msg_chunks = []
        deg_chunks = []
        for ci in range(NCH):
            i0 = ci * CH
            pxi = spos_ref[i0:i0 + CH, 0:1]
            pyi = spos_ref[i0:i0 + CH, 1:2]
            pzi = spos_ref[i0:i0 + CH, 2:3]
            dx = sposT_ref[0:1, :] - pxi
            dy = sposT_ref[1:2, :] - pyi
            dz = sposT_ref[2:3, :] - pzi
            d2 = dx * dx + dy * dy + dz * dz
            rowid = lax.broadcasted_iota(jnp.int32, (CH, MP), 0) + i0
            colid = lax.broadcasted_iota(jnp.int32, (CH, MP), 1)
            mask = (d2 < r2) & (rowid != colid)
            maskf = mask.astype(jnp.float32)
            ux = jnp.clip(dx * inv2r + 0.5, 0.0, 1.0)
            uy = jnp.clip(dy * inv2r + 0.5, 0.0, 1.0)
            uz = jnp.clip(dz * inv2r + 0.5, 0.0, 1.0)
            bs = []
            for (b0, b1_, b2_) in _BITS:
                w = (ux if b0 else 1.0 - ux)
                w = w * (uy if b1_ else 1.0 - uy)
                w = w * (uz if b2_ else 1.0 - uz)
                bs.append(w * maskf)
            b2cat = jnp.concatenate(bs, axis=1)
            msg_chunks.append(jnp.dot(b2cat, xwcat,
                                      preferred_element_type=jnp.float32))
            deg_chunks.append(jnp.sum(maskf, axis=1, keepdims=True))
        msg = jnp.concatenate(msg_chunks, axis=0)
        deg = jnp.concatenate(deg_chunks, axis=0)
        return msg, deg

    def bn(h, g_ref, be_ref):
        hm = jnp.where(vmask_col, h, 0.0)
        mu = jnp.sum(hm, axis=0, keepdims=True) * (1.0 / M)
        d = h - mu
        dm = jnp.where(vmask_col, d * d, 0.0)
        var = jnp.sum(dm, axis=0, keepdims=True) * (1.0 / M)
        return d * jax.lax.rsqrt(var + 1e-5) * g_ref[...] + be_ref[...]

    msg6, deg = dense_msg(sx, w6_ref)
    degc = jnp.maximum(deg, 1.0)
    h6 = msg6 / degc + jnp.dot(sx, root6_ref[...],
                               preferred_element_type=jnp.float32) + b6_ref[...]
    h6 = bn(_elu(h6), g6_ref, be6_ref)

    msg7, _ = dense_msg(h6, w7_ref)
    h7 = msg7 / degc + jnp.dot(h6, root7_ref[...],
                               preferred_element_type=jnp.float32) + b7_ref[...]
    h7 = bn(_elu(h7), g7_ref, be7_ref)

    h2 = h7 + sx
    g = jnp.max(jnp.where(vmask_col, h2, -jnp.inf), axis=0, keepdims=True)
    logits = jnp.dot(g, fcw_ref[...],
                     preferred_element_type=jnp.float32) + fcb_ref[...]
    out_ref[...] = jnp.zeros((8, 128), jnp.float32)
    out_ref[0:1, 0:NUM_CLASSES] = logits


def _pooled_stage(spos, sx, p6, bn6, p7, bn7, fc_w, fc_b):
    sposp = jnp.concatenate(
        [spos, jnp.zeros((M, 1), jnp.float32)], axis=1)
    sposp = jnp.concatenate(
        [sposp, jnp.full((MP - M, 4), 1e6, jnp.float32)], axis=0)
    sxp = jnp.concatenate([sx, jnp.zeros((MP - M, DIM3), sx.dtype)], axis=0)
    out = pl.pallas_call(
        _pooled_kernel,
        out_shape=jax.ShapeDtypeStruct((8, 128), jnp.float32),
    )(sposp, sposp.T, sxp,
      p6['W'], p6['root'], p6['b'][None, :], bn6['gamma'][None, :],
      bn6['beta'][None, :],
      p7['W'], p7['root'], p7['b'][None, :], bn7['gamma'][None, :],
      bn7['beta'][None, :],
      fc_w, fc_b[None, :])
    return out[0:1, 0:NUM_CLASSES]


NPAD = 10240
FPS_R, FPS_C = 80, 128


def _fps_kernel(px_ref, py_ref, pz_ref, out_ref):
    px = px_ref[...]
    py = py_ref[...]
    pz = pz_ref[...]
    flat = (lax.broadcasted_iota(jnp.int32, (FPS_R, FPS_C), 0) * FPS_C
            + lax.broadcasted_iota(jnp.int32, (FPS_R, FPS_C), 1))
    valid = flat < N
    flat_o = (lax.broadcasted_iota(jnp.int32, (8, 128), 0) * 128
              + lax.broadcasted_iota(jnp.int32, (8, 128), 1))
    dist0 = jnp.where(valid, jnp.inf, -jnp.inf)
    cx0 = px[0, 0]
    cy0 = py[0, 0]
    cz0 = pz[0, 0]
    idxs0 = jnp.zeros((8, 128), jnp.int32)

    def body(i, carry):
        dist, cx, cy, cz, idxs = carry
        dx = px - cx
        dy = py - cy
        dz = pz - cz
        d = dx * dx + dy * dy + dz * dz
        dist = jnp.minimum(dist, d)
        m = jnp.max(dist)
        jsel = jnp.where(dist == m, flat, jnp.int32(2**30))
        j = jnp.min(jsel)
        onehot = flat == j
        ohf = onehot.astype(jnp.float32)
        ncx = jnp.sum(px * ohf)
        ncy = jnp.sum(py * ohf)
        ncz = jnp.sum(pz * ohf)
        idxs = jnp.where(flat_o == i + 1, j, idxs)
        return dist, ncx, ncy, ncz, idxs

    _, _, _, _, idxs = lax.fori_loop(0, M - 1, body,
                                     (dist0, cx0, cy0, cz0, idxs0))
    out_ref[...] = idxs


def _fps_pallas(pos):
    pad = jnp.zeros((NPAD - N,), jnp.float32)
    px = jnp.concatenate([pos[:, 0], pad]).reshape(FPS_R, FPS_C)
    py = jnp.concatenate([pos[:, 1], pad]).reshape(FPS_R, FPS_C)
    pz = jnp.concatenate([pos[:, 2], pad]).reshape(FPS_R, FPS_C)
    out = pl.pallas_call(
        _fps_kernel,
        out_shape=jax.ShapeDtypeStruct((8, 128), jnp.int32),
    )(px, py, pz)
    return out.reshape(-1)[:M]


def _basis_edges(u):
    u = jnp.clip(u, 0.0, 1.0)
    ue = u[..., None, :]
    return jnp.prod(jnp.where(_BITS_F > 0, ue, 1.0 - ue), axis=-1)


def _conv_edges(x, src, dst, b, p, n_nodes):
    xs = x[src]
    msg = jnp.zeros((xs.shape[0], p['W'].shape[2]), dtype=x.dtype)
    for k in range(K3):
        msg = msg + b[:, k:k + 1] * (xs @ p['W'][k])
    agg = jax.ops.segment_sum(msg, dst, num_segments=n_nodes)
    cnt = jax.ops.segment_sum(jnp.ones((xs.shape[0],), dtype=x.dtype), dst,
                              num_segments=n_nodes)
    return agg / jnp.maximum(cnt, 1.0)[:, None] + x @ p['root'] + p['b']


def _bn_full(x, p, eps=1e-5):
    mu = x.mean(axis=0)
    var = x.var(axis=0)
    return (x - mu) / jnp.sqrt(var + eps) * p['gamma'] + p['beta']


def _fps_jax(pos, m):
    n = pos.shape[0]
    idxs = jnp.zeros((m,), dtype=jnp.int32)
    dist = jnp.full((n,), jnp.inf, dtype=pos.dtype)

    def body(i, carry):
        dist, idxs = carry
        last = pos[idxs[i]]
        d = jnp.sum((pos - last) ** 2, axis=1)
        dist = jnp.minimum(dist, d)
        idxs = idxs.at[i + 1].set(jnp.argmax(dist).astype(jnp.int32))
        return dist, idxs

    dist, idxs = jax.lax.fori_loop(0, m - 1, body, (dist, idxs))
    return idxs


def kernel(x, pos, batch, edge_index, params):
    src, dst = edge_index[0], edge_index[1]
    u1 = (pos[src] - pos[dst]) / (2.0 * NODE_RADIUS) + 0.5
    b1 = _basis_edges(u1)
    h = params['emb'][x]
    h = _bn_full(jax.nn.elu(_conv_edges(h, src, dst, b1, params['conv1'], N)),
                 params['bn1'])
    h = _bn_full(jax.nn.elu(_conv_edges(h, src, dst, b1, params['conv2'], N)),
                 params['bn2'])
    h_sc = h
    h = _bn_full(jax.nn.elu(_conv_edges(h, src, dst, b1, params['conv3'], N)),
                 params['bn3'])
    h = _bn_full(jax.nn.elu(_conv_edges(h, src, dst, b1, params['conv4'], N)),
                 params['bn4'])
    h = h + h_sc
    h = _bn_full(jax.nn.elu(_conv_edges(h, src, dst, b1, params['conv5'], N)),
                 params['bn5'])
    idx = _fps_pallas(lax.stop_gradient(pos))
    sx = h[idx]
    spos = pos[idx]
    return _pooled_stage(spos, sx, params['conv6'], params['bn6'],
                         params['conv7'], params['bn7'],
                         params['fc_w'], params['fc_b'])
